# Initial kernel scaffold; baseline (speedup 1.0000x reference)
import jax
import jax.numpy as jnp
from jax.experimental import pallas as pl


def kernel(rpn_probs, rpn_bbox, anchors):
    raise NotImplementedError("write your pallas kernel here")


# speedup vs baseline: 22.0387x; 22.0387x over previous
import functools

import jax
import jax.numpy as jnp
import numpy as np
from jax.experimental import pallas as pl
from jax.experimental.pallas import tpu as pltpu

N_ANCHORS = 20000
LANES = 128
ROWS = 160
NPAD = ROWS * LANES
PRE_NMS = 6000
N_OUT = 1000
IOU_THR = 0.7
NEG = np.float32(-1e38)


def _proposal_kernel(scores_ref, geom_ref, out_ref,
                     masked_ref, y1_ref, x1_ref, y2_ref, x2_ref, area_ref):
    scores = scores_ref[0]

    ay1 = geom_ref[0, 0]
    ax1 = geom_ref[0, 1]
    ay2 = geom_ref[0, 2]
    ax2 = geom_ref[0, 3]
    dy = geom_ref[0, 4] * np.float32(0.1)
    dx = geom_ref[0, 5] * np.float32(0.1)
    dh = geom_ref[0, 6] * np.float32(0.2)
    dw = geom_ref[0, 7] * np.float32(0.2)
    height = ay2 - ay1
    width = ax2 - ax1
    center_y = ay1 + np.float32(0.5) * height
    center_x = ax1 + np.float32(0.5) * width
    center_y = center_y + dy * height
    center_x = center_x + dx * width
    height = height * jnp.exp(dh)
    width = width * jnp.exp(dw)
    y1 = center_y - np.float32(0.5) * height
    x1 = center_x - np.float32(0.5) * width
    y2 = y1 + height
    x2 = x1 + width
    one = np.float32(1.0)
    zero = np.float32(0.0)
    y1 = jnp.maximum(jnp.minimum(y1, one), zero)
    x1 = jnp.maximum(jnp.minimum(x1, one), zero)
    y2 = jnp.maximum(jnp.minimum(y2, one), zero)
    x2 = jnp.maximum(jnp.minimum(x2, one), zero)
    y1_ref[...] = y1
    x1_ref[...] = x1
    y2_ref[...] = y2
    x2_ref[...] = x2
    area_ref[...] = (y2 - y1) * (x2 - x1)

    bits = jax.lax.bitcast_convert_type(scores, jnp.int32)

    def count_ge(v):
        return jnp.sum((bits >= v).astype(jnp.int32))

    def bs_body(_, state):
        lo, hi = state
        mid = lo + (hi - lo) // 2
        ge = count_ge(mid) >= PRE_NMS
        return (jnp.where(ge, mid, lo), jnp.where(ge, hi, mid))

    lo, hi = jax.lax.fori_loop(
        0, 31, bs_body,
        (jnp.int32(0), jnp.int32(np.int32(0x7F800000))))
    vstar = lo
    count_gt = jnp.sum((bits > vstar).astype(jnp.int32))
    k_ties = PRE_NMS - count_gt

    row_iota = jax.lax.broadcasted_iota(jnp.int32, (ROWS, LANES), 0)
    col_iota = jax.lax.broadcasted_iota(jnp.int32, (ROWS, LANES), 1)
    idx2d = row_iota * LANES + col_iota
    is_tie = bits == vstar

    def count_tie_lt(i):
        return jnp.sum((is_tie & (idx2d < i)).astype(jnp.int32))

    def bs2_body(_, state):
        lo2, hi2 = state
        mid = lo2 + (hi2 - lo2) // 2
        ge = count_tie_lt(mid) >= k_ties
        return (jnp.where(ge, lo2, mid), jnp.where(ge, mid, hi2))

    lo2, hi2 = jax.lax.fori_loop(
        0, 15, bs2_body, (jnp.int32(0), jnp.int32(NPAD)))
    istar = hi2

    active = (bits > vstar) | (is_tie & (idx2d < istar))
    masked_ref[...] = jnp.where(active, scores, NEG)

    out_ref[...] = jnp.zeros_like(out_ref)
    out_iota = (jax.lax.broadcasted_iota(jnp.int32, (8, LANES), 0) * LANES
                + jax.lax.broadcasted_iota(jnp.int32, (8, LANES), 1))
    thr = np.float32(IOU_THR)
    big = jnp.int32(2 ** 30)

    def nms_body(i, carry):
        masked = masked_ref[...]
        m = jnp.max(masked)

        @pl.when(m >= zero)
        def _():
            sel = masked == m
            j = jnp.min(jnp.where(sel, idx2d, big))
            selj = idx2d == j
            cy1 = y1_ref[...]
            cx1 = x1_ref[...]
            cy2 = y2_ref[...]
            cx2 = x2_ref[...]
            car = area_ref[...]
            fz = jnp.float32(0.0)
            by1 = jnp.sum(jnp.where(selj, cy1, fz))
            bx1 = jnp.sum(jnp.where(selj, cx1, fz))
            by2 = jnp.sum(jnp.where(selj, cy2, fz))
            bx2 = jnp.sum(jnp.where(selj, cx2, fz))
            barea = jnp.sum(jnp.where(selj, car, fz))
            yy1 = jnp.maximum(by1, cy1)
            xx1 = jnp.maximum(bx1, cx1)
            yy2 = jnp.minimum(by2, cy2)
            xx2 = jnp.minimum(bx2, cx2)
            inter = jnp.maximum(yy2 - yy1, zero) * jnp.maximum(xx2 - xx1, zero)
            union = barea + car - inter
            iou = jnp.where(union > zero, inter / union, zero)
            suppress = (iou > thr) | selj
            masked_ref[...] = jnp.where(suppress, NEG, masked)
            selo = out_iota == i
            out_ref[0, 0] = jnp.where(selo, by1, out_ref[0, 0])
            out_ref[0, 1] = jnp.where(selo, bx1, out_ref[0, 1])
            out_ref[0, 2] = jnp.where(selo, by2, out_ref[0, 2])
            out_ref[0, 3] = jnp.where(selo, bx2, out_ref[0, 3])

        return carry

    jax.lax.fori_loop(0, N_OUT, nms_body, jnp.int32(0))


@jax.jit
def kernel(rpn_probs, rpn_bbox, anchors):
    batch = rpn_probs.shape[0]
    scores = rpn_probs[:, :, 1]
    scores = jnp.pad(scores, ((0, 0), (0, NPAD - N_ANCHORS)),
                     constant_values=-1.0)
    scores = scores.reshape(batch, ROWS, LANES)
    geom = jnp.concatenate(
        [anchors.transpose(0, 2, 1), rpn_bbox.transpose(0, 2, 1)], axis=1)
    geom = jnp.pad(geom, ((0, 0), (0, 0), (0, NPAD - N_ANCHORS)))
    geom = geom.reshape(batch, 8, ROWS, LANES)

    out = pl.pallas_call(
        _proposal_kernel,
        grid=(batch,),
        in_specs=[
            pl.BlockSpec((1, ROWS, LANES), lambda b: (b, 0, 0)),
            pl.BlockSpec((1, 8, ROWS, LANES), lambda b: (b, 0, 0, 0)),
        ],
        out_specs=pl.BlockSpec((1, 4, 8, LANES), lambda b: (b, 0, 0, 0)),
        out_shape=jax.ShapeDtypeStruct((batch, 4, 8, LANES), jnp.float32),
        scratch_shapes=[
            pltpu.VMEM((ROWS, LANES), jnp.float32),
            pltpu.VMEM((ROWS, LANES), jnp.float32),
            pltpu.VMEM((ROWS, LANES), jnp.float32),
            pltpu.VMEM((ROWS, LANES), jnp.float32),
            pltpu.VMEM((ROWS, LANES), jnp.float32),
            pltpu.VMEM((ROWS, LANES), jnp.float32),
        ],
    )(scores, geom)

    out = out.reshape(batch, 4, 8 * LANES)[:, :, :N_OUT]
    return out.transpose(0, 2, 1)
